# baseline (speedup 1.0000x reference)
import functools

import jax
import jax.numpy as jnp
from jax import lax
from jax.experimental import pallas as pl
from jax.experimental.pallas import tpu as pltpu
from jax.experimental.pallas import tpu_sc as plsc

EMB_D = 64
IDX_CHUNK = 128


def _make_sc_kernel(num_workers, b_per_w, n_chunks):
    mesh = plsc.VectorSubcoreMesh(core_axis_name="c", subcore_axis_name="s")
    num_cores = 2

    @functools.partial(
        pl.kernel,
        out_type=jax.ShapeDtypeStruct((16384, 128), jnp.float32),
        mesh=mesh,
        scratch_types=[
            pltpu.VMEM((n_chunks, IDX_CHUNK), jnp.int32),
            pltpu.VMEM((b_per_w, 128), jnp.float32),
            pltpu.SemaphoreType.DMA,
        ],
    )
    def emb_combine(xw_hbm, poi_hbm, loc_hbm, out_hbm, idx_v, g_v, sem):
        wid = lax.axis_index("s") * num_cores + lax.axis_index("c")
        base = wid * b_per_w
        pltpu.sync_copy(xw_hbm.at[wid], idx_v)
        copies = []
        for j in range(n_chunks):
            rows = pl.ds(j * IDX_CHUNK, IDX_CHUNK)
            copies.append(
                pltpu.async_copy(poi_hbm.at[idx_v.at[j]], g_v.at[rows], sem))
            copies.append(
                pltpu.async_copy(loc_hbm.at[idx_v.at[j]], g_v.at[rows], sem))
        for c in copies:
            c.wait()
        pltpu.sync_copy(g_v, out_hbm.at[pl.ds(base, b_per_w)])

    return emb_combine


def kernel(x, poi_table, loc_table):
    b = x.shape[0]
    info = plsc.get_sparse_core_info()
    num_workers = info.num_cores * info.num_subcores
    b_per_w = b // num_workers
    n_chunks = b_per_w // IDX_CHUNK
    xw = (x.astype(jnp.int32) >> 1).reshape(num_workers, n_chunks, IDX_CHUNK)
    poiw = poi_table.reshape(50000, 128)
    locw = loc_table.reshape(50000, 128)
    out = _make_sc_kernel(num_workers, b_per_w, n_chunks)(xw, poiw, locw)
    return out


# speedup vs baseline: 1.0808x; 1.0808x over previous
import functools

import jax
import jax.numpy as jnp
from jax import lax
from jax.experimental import pallas as pl
from jax.experimental.pallas import tpu as pltpu
from jax.experimental.pallas import tpu_sc as plsc

OUT_D = 128
IDX_CHUNK = 128


def _make_sc_kernel(num_workers, b_per_w, n_chunks):
    mesh = plsc.VectorSubcoreMesh(core_axis_name="c", subcore_axis_name="s")
    num_cores = 2

    @functools.partial(
        pl.kernel,
        out_type=jax.ShapeDtypeStruct((num_workers * b_per_w, OUT_D),
                                      jnp.float32),
        mesh=mesh,
        scratch_types=[
            pltpu.VMEM((n_chunks, IDX_CHUNK), jnp.int32),
            pltpu.VMEM((b_per_w, OUT_D), jnp.float32),
            [pltpu.SemaphoreType.DMA] * 4,
            pltpu.SemaphoreType.DMA,
        ],
    )
    def emb_combine(x_hbm, poi_hbm, loc_hbm, out_hbm, idx_v, g_v, gsems,
                    addsem):
        wid = lax.axis_index("s") * num_cores + lax.axis_index("c")
        base = wid * b_per_w
        pltpu.sync_copy(x_hbm.at[wid], idx_v)
        gathers = []
        for j in range(n_chunks):
            rows = pl.ds(j * IDX_CHUNK, IDX_CHUNK)
            gathers.append(
                pltpu.async_copy(poi_hbm.at[idx_v.at[j]], g_v.at[rows],
                                 gsems[j]))
        adds = []
        for j in range(n_chunks):
            rows = pl.ds(j * IDX_CHUNK, IDX_CHUNK)
            gathers[j].wait()
            adds.append(
                pltpu.async_copy(loc_hbm.at[idx_v.at[j]], g_v.at[rows],
                                 addsem, add=True))
        for c in adds:
            c.wait()
        pltpu.sync_copy(g_v, out_hbm.at[pl.ds(base, b_per_w)])

    return emb_combine


def kernel(x, poi_table, loc_table):
    b = x.shape[0]
    emb_d = poi_table.shape[1]
    info = plsc.get_sparse_core_info()
    num_workers = info.num_cores * info.num_subcores
    b_per_w = b // num_workers
    n_chunks = b_per_w // IDX_CHUNK
    x2 = x.astype(jnp.int32).reshape(num_workers, n_chunks, IDX_CHUNK)
    poipad = jnp.pad(poi_table, ((0, 0), (0, OUT_D - emb_d)))
    locpad = jnp.pad(loc_table, ((0, 0), (OUT_D - emb_d, 0)))
    return _make_sc_kernel(num_workers, b_per_w, n_chunks)(
        x2, poipad, locpad)


# speedup vs baseline: 1.1937x; 1.1045x over previous
import functools

import jax
import jax.numpy as jnp
from jax import lax
from jax.experimental import pallas as pl
from jax.experimental.pallas import tpu as pltpu
from jax.experimental.pallas import tpu_sc as plsc

OUT_D = 128
IDX_CHUNK = 128


def _make_sc_kernel(num_workers, b_per_w, n_chunks):
    mesh = plsc.VectorSubcoreMesh(core_axis_name="c", subcore_axis_name="s")
    num_cores = 2

    @functools.partial(
        pl.kernel,
        out_type=jax.ShapeDtypeStruct((num_workers * b_per_w, OUT_D),
                                      jnp.float32),
        mesh=mesh,
        scratch_types=[
            pltpu.VMEM((n_chunks, IDX_CHUNK), jnp.int32),
            pltpu.VMEM((b_per_w, OUT_D), jnp.float32),
            pltpu.SemaphoreType.DMA,
        ],
    )
    def emb_combine(x_hbm, comb_hbm, out_hbm, idx_v, g_v, sem):
        wid = lax.axis_index("s") * num_cores + lax.axis_index("c")
        base = wid * b_per_w
        pltpu.sync_copy(x_hbm.at[wid], idx_v)
        gathers = []
        for j in range(n_chunks):
            rows = pl.ds(j * IDX_CHUNK, IDX_CHUNK)
            gathers.append(
                pltpu.async_copy(comb_hbm.at[idx_v.at[j]], g_v.at[rows],
                                 sem))
        for c in gathers:
            c.wait()
        pltpu.sync_copy(g_v, out_hbm.at[pl.ds(base, b_per_w)])

    return emb_combine


def kernel(x, poi_table, loc_table):
    b = x.shape[0]
    info = plsc.get_sparse_core_info()
    num_workers = info.num_cores * info.num_subcores
    b_per_w = b // num_workers
    n_chunks = b_per_w // IDX_CHUNK
    x2 = x.astype(jnp.int32).reshape(num_workers, n_chunks, IDX_CHUNK)
    comb = jnp.concatenate([poi_table, loc_table], axis=1)
    return _make_sc_kernel(num_workers, b_per_w, n_chunks)(x2, comb)


# speedup vs baseline: 1.4034x; 1.1757x over previous
import functools

import jax
import jax.numpy as jnp
from jax import lax
from jax.experimental import pallas as pl
from jax.experimental.pallas import tpu as pltpu
from jax.experimental.pallas import tpu_sc as plsc

OUT_D = 128
IDX_CHUNK = 128


def _make_sc_kernel(num_workers, b_per_w, n_chunks):
    mesh = plsc.VectorSubcoreMesh(core_axis_name="c", subcore_axis_name="s")
    num_cores = 2

    @functools.partial(
        pl.kernel,
        out_type=jax.ShapeDtypeStruct((num_workers * b_per_w, OUT_D),
                                      jnp.float32),
        mesh=mesh,
        scratch_types=[
            pltpu.VMEM((n_chunks, IDX_CHUNK), jnp.int32),
            pltpu.VMEM((b_per_w, OUT_D), jnp.float32),
            pltpu.SemaphoreType.DMA,
        ],
    )
    def emb_combine(x_hbm, comb_hbm, out_hbm, idx_v, g_v, sem):
        wid = lax.axis_index("s") * num_cores + lax.axis_index("c")
        base = wid * b_per_w
        pltpu.sync_copy(x_hbm.at[wid], idx_v)
        gathers = []
        for j in range(n_chunks):
            rows = pl.ds(j * IDX_CHUNK, IDX_CHUNK)
            gathers.append(
                pltpu.async_copy(comb_hbm.at[idx_v.at[j]], g_v.at[rows],
                                 sem))
        for c in gathers:
            c.wait()
        pltpu.sync_copy(g_v, out_hbm.at[pl.ds(base, b_per_w)])

    return emb_combine


def kernel(x, poi_table, loc_table):
    b = x.shape[0]
    info = plsc.get_sparse_core_info()
    num_workers = info.num_cores * info.num_subcores
    b_per_w = b // num_workers
    n_chunks = b_per_w // IDX_CHUNK
    x2 = x.astype(jnp.int32).reshape(num_workers, n_chunks, IDX_CHUNK)
    comb = jnp.swapaxes(jnp.stack([poi_table, loc_table]), 0, 1).reshape(
        poi_table.shape[0], 2 * poi_table.shape[1])
    return _make_sc_kernel(num_workers, b_per_w, n_chunks)(x2, comb)
